# baseline (speedup 1.0000x reference)
import functools

import jax
import jax.numpy as jnp
from jax import lax
from jax.experimental import pallas as pl
from jax.experimental.pallas import tpu as pltpu
from jax.experimental.pallas import tpu_sc as plsc

N = 50000
E = 800000
NODE_F = 13
EDGE_F = 2
FACE_D = 8
HID = 64
HEADS = 4
OUT_C = HID // HEADS
LAYERS = 3
LAT = 32

NC = 2
NS = 16
CH = 128

ROWS = 6272
E_PAD = ROWS * CH
ROWS_A = ROWS // (NC * NS)
ROWS_B = ROWS // NS
N_PAD = 50176
NPT = N_PAD // NS

SCKA = 7

RB = 1792
RBE = ROWS // 16


def _k0_body(x_ref, ft_ref, emb_ref, w_ref, b_ref, out_ref):
    xb = x_ref[...]
    ft = ft_ref[...]
    emb = emb_ref[...]
    fe = jnp.where(ft == 0, emb[0][None, :],
                   jnp.where(ft == 1, emb[1][None, :], emb[2][None, :]))
    xc = jnp.concatenate([xb, fe], axis=1)
    h = jnp.maximum(xc @ w_ref[...] + b_ref[...], 0.0)
    for k in range(HEADS):
        out_ref[k] = h[:, 16 * k:16 * k + 16]


def _node_init(x_p, ft_p, emb, w, b):
    return pl.pallas_call(
        _k0_body,
        grid=(N_PAD // RB,),
        in_specs=[
            pl.BlockSpec((RB, NODE_F), lambda i: (i, 0)),
            pl.BlockSpec((RB, 1), lambda i: (i, 0)),
            pl.BlockSpec((3, FACE_D), lambda i: (0, 0)),
            pl.BlockSpec((NODE_F + FACE_D, HID), lambda i: (0, 0)),
            pl.BlockSpec((1, HID), lambda i: (0, 0)),
        ],
        out_specs=pl.BlockSpec((HEADS, RB, 16), lambda i: (0, i, 0)),
        out_shape=jax.ShapeDtypeStruct((HEADS, N_PAD, 16), jnp.float32),
    )(x_p, ft_p, emb, w, b)


def _ke_body(ea_ref, wt_ref, b_ref, awt_ref, o0_ref, o1_ref, o2_ref):
    eh = jnp.maximum(wt_ref[...] @ ea_ref[...] + b_ref[...], 0.0)
    a = awt_ref[...] @ eh
    for l, o in enumerate((o0_ref, o1_ref, o2_ref)):
        o[...] = a[l * HEADS:(l + 1) * HEADS].reshape(HEADS, RBE, CH)


def _edge_logits(ea_t, wt, b, awt):
    return pl.pallas_call(
        _ke_body,
        grid=(ROWS // RBE,),
        in_specs=[
            pl.BlockSpec((EDGE_F, RBE * CH), lambda i: (0, i)),
            pl.BlockSpec((HID, EDGE_F), lambda i: (0, 0)),
            pl.BlockSpec((HID, 1), lambda i: (0, 0)),
            pl.BlockSpec((LAYERS * HEADS, HID), lambda i: (0, 0)),
        ],
        out_specs=[pl.BlockSpec((HEADS, RBE, CH), lambda i: (0, i, 0))] * 3,
        out_shape=[jax.ShapeDtypeStruct((HEADS, ROWS, CH), jnp.float32)] * 3,
    )(ea_t, wt, b, awt)


def _k1_body(hh_ref, pb_ref, w_ref, as_ref, ad_ref, xhh_ref, s_ref, d_ref):
    h = jnp.concatenate([hh_ref[k] for k in range(HEADS)], axis=1)
    h = jnp.maximum(h + pb_ref[...], 0.0)
    xh = h @ w_ref[...]
    for k in range(HEADS):
        xhh_ref[k] = xh[:, 16 * k:16 * k + 16]
    s_ref[...] = (h @ as_ref[...]).T
    d_ref[...] = (h @ ad_ref[...]).T


def _layer_proj(hh, pb, w, a_s, a_d):
    return pl.pallas_call(
        _k1_body,
        grid=(N_PAD // RB,),
        in_specs=[
            pl.BlockSpec((HEADS, RB, 16), lambda i: (0, i, 0)),
            pl.BlockSpec((1, HID), lambda i: (0, 0)),
            pl.BlockSpec((HID, HID), lambda i: (0, 0)),
            pl.BlockSpec((HID, HEADS), lambda i: (0, 0)),
            pl.BlockSpec((HID, HEADS), lambda i: (0, 0)),
        ],
        out_specs=[
            pl.BlockSpec((HEADS, RB, 16), lambda i: (0, i, 0)),
            pl.BlockSpec((HEADS, RB), lambda i: (0, i)),
            pl.BlockSpec((HEADS, RB), lambda i: (0, i)),
        ],
        out_shape=[
            jax.ShapeDtypeStruct((HEADS, N_PAD, 16), jnp.float32),
            jax.ShapeDtypeStruct((HEADS, N_PAD), jnp.float32),
            jax.ShapeDtypeStruct((HEADS, N_PAD), jnp.float32),
        ],
    )(hh, pb, w, a_s, a_d)


def _kf_body(h_ref, b_ref, mw_ref, mb_ref, lw_ref, lb_ref, mu_ref, lv_ref):
    rid = lax.broadcasted_iota(jnp.int32, (N_PAD, 1), 0)
    h = jnp.maximum(h_ref[...] + b_ref[...], 0.0)
    h = jnp.where(rid < N, h, 0.0)
    hm = jnp.sum(h, axis=0, keepdims=True) * (1.0 / N)
    mu_ref[...] = hm @ mw_ref[...] + mb_ref[...]
    lv_ref[...] = hm @ lw_ref[...] + lb_ref[...]


def _final(hh, b, mw, mb, lw, lb):
    return pl.pallas_call(
        _kf_body,
        out_shape=(
            jax.ShapeDtypeStruct((1, LAT), jnp.float32),
            jax.ShapeDtypeStruct((1, LAT), jnp.float32),
        ),
    )(hh, b, mw, mb, lw, lb)


_MESH = plsc.VectorSubcoreMesh(core_axis_name="c", subcore_axis_name="s")
_SC_PARAMS = pltpu.CompilerParams(use_tc_tiling_on_sc=False)


@functools.partial(
    pl.kernel,
    out_type=(
        jax.ShapeDtypeStruct((HEADS, ROWS, CH), jnp.float32),
        jax.ShapeDtypeStruct((HEADS, N_PAD), jnp.float32),
    ),
    mesh=_MESH,
    compiler_params=_SC_PARAMS,
    scratch_types=[
        pltpu.VMEM((SCKA, CH), jnp.int32),
        pltpu.VMEM((SCKA, CH), jnp.int32),
        [pltpu.VMEM((SCKA, CH), jnp.float32)] * 2,
        [pltpu.VMEM((SCKA, CH), jnp.float32)] * 2,
        pltpu.VMEM((CH,), jnp.float32),
        pltpu.VMEM((CH,), jnp.float32),
        pltpu.VMEM_SHARED((2, N_PAD), jnp.float32),
        pltpu.VMEM_SHARED((2, N_PAD), jnp.float32),
        pltpu.VMEM_SHARED((2, N_PAD), jnp.float32),
        pltpu.SemaphoreType.DMA,
        pltpu.SemaphoreType.DMA,
    ],
)
def _sc_pass_a(src_hbm, dst_hbm, ae_hbm, asrc_hbm, adst_hbm, z4_hbm,
               ex_hbm, den_hbm,
               isb, idb, aeb, exb, gs_v, gd_v,
               asrc_sh, adst_sh, den_sh, sem_sc, sem_ex):
    c = lax.axis_index("c")
    s = lax.axis_index("s")
    row0 = s * NPT
    for hh in range(2):
        pltpu.sync_copy(asrc_hbm.at[c * 2 + hh, pl.ds(row0, NPT)],
                        asrc_sh.at[hh, pl.ds(row0, NPT)])
        pltpu.sync_copy(adst_hbm.at[c * 2 + hh, pl.ds(row0, NPT)],
                        adst_sh.at[hh, pl.ds(row0, NPT)])
        pltpu.sync_copy(z4_hbm.at[hh, pl.ds(row0, NPT)],
                        den_sh.at[hh, pl.ds(row0, NPT)])
    plsc.subcore_barrier()

    nsa = ROWS_B // SCKA

    @pl.loop(0, nsa)
    def _(j):
        r0 = s * ROWS_B + j * SCKA
        pltpu.sync_copy(src_hbm.at[pl.ds(r0, SCKA)], isb)
        pltpu.sync_copy(dst_hbm.at[pl.ds(r0, SCKA)], idb)
        for hh in range(2):
            pltpu.sync_copy(ae_hbm.at[c * 2 + hh, pl.ds(r0, SCKA)], aeb[hh])
        sdesc = []
        for k in range(SCKA):
            for hh in range(2):
                pltpu.sync_copy(asrc_sh.at[hh].at[isb.at[k]], gs_v)
                pltpu.sync_copy(adst_sh.at[hh].at[idb.at[k]], gd_v)
                for i in range(CH // 16):
                    a = (gs_v[pl.ds(16 * i, 16)] + gd_v[pl.ds(16 * i, 16)]
                         + aeb[hh][k, pl.ds(16 * i, 16)])
                    a = jnp.where(a >= 0.0, a, 0.2 * a)
                    exb[hh][k, pl.ds(16 * i, 16)] = jnp.exp(a)
                sdesc.append(pltpu.async_copy(
                    exb[hh].at[k], den_sh.at[hh].at[idb.at[k]], sem_sc,
                    add=True))
                if len(sdesc) > 2:
                    sdesc.pop(0).wait()
        edesc = [pltpu.async_copy(exb[hh],
                                  ex_hbm.at[c * 2 + hh, pl.ds(r0, SCKA)],
                                  sem_ex)
                 for hh in range(2)]
        for d in sdesc:
            d.wait()
        for d in edesc:
            d.wait()

    plsc.subcore_barrier()
    for hh in range(2):
        pltpu.sync_copy(den_sh.at[hh, pl.ds(row0, NPT)],
                        den_hbm.at[c * 2 + hh, pl.ds(row0, NPT)])


SCK = 8
NSC = ROWS_B // SCK
SCKA = 7


@functools.partial(
    pl.kernel,
    out_type=jax.ShapeDtypeStruct((HEADS, N_PAD, 16), jnp.float32),
    mesh=_MESH,
    compiler_params=_SC_PARAMS,
    scratch_types=[
        pltpu.VMEM((SCK, CH), jnp.int32),
        pltpu.VMEM((SCK, CH), jnp.int32),
        pltpu.VMEM((SCK, CH), jnp.float32),
        pltpu.VMEM((SCK, CH), jnp.float32),
        pltpu.VMEM((SCK * CH, 16), jnp.float32),
        pltpu.VMEM_SHARED((N_PAD,), jnp.float32),
        pltpu.VMEM_SHARED((N_PAD, 16), jnp.float32),
        pltpu.SemaphoreType.DMA,
        pltpu.SemaphoreType.DMA,
    ],
)
def _sc_pass_b(src_hbm, dst_hbm, ex_hbm, den_hbm, xh_hbm, z16_hbm,
               out_hbm,
               isb, idb, exb, dnb, rows, den_sh, out_sh,
               sem_sc, sem_ex):
    c = lax.axis_index("c")
    s = lax.axis_index("s")
    row0 = s * NPT
    for half in range(2):
        hd = c * 2 + half
        pltpu.sync_copy(den_hbm.at[hd, pl.ds(row0, NPT)],
                        den_sh.at[pl.ds(row0, NPT)])
        pltpu.sync_copy(z16_hbm.at[pl.ds(row0, NPT)],
                        out_sh.at[pl.ds(row0, NPT)])
        plsc.subcore_barrier()

        @pl.loop(0, NSC)
        def _(j):
            r0 = s * ROWS_B + j * SCK
            pltpu.sync_copy(src_hbm.at[pl.ds(r0, SCK)], isb)
            pltpu.sync_copy(dst_hbm.at[pl.ds(r0, SCK)], idb)
            ed = pltpu.async_copy(ex_hbm.at[hd, pl.ds(r0, SCK)], exb, sem_ex)
            ed.wait()

            sdesc = {}
            for k in range(SCK):
                pltpu.sync_copy(den_sh.at[idb.at[k]], dnb.at[k])
                pltpu.sync_copy(xh_hbm.at[hd].at[isb.at[k]],
                                rows.at[pl.ds(k * CH, CH)])
                for i in range(CH // 16):
                    dnb[k, pl.ds(16 * i, 16)] = (exb[k, pl.ds(16 * i, 16)]
                                                 / dnb[k, pl.ds(16 * i, 16)])
                for e in range(CH):
                    wrow = dnb[k, pl.ds((e // 16) * 16, 16)]
                    wsp = jnp.broadcast_to(
                        lax.slice(wrow, (e % 16,), (e % 16 + 1,)), (16,))
                    rr = k * CH + e
                    rows[rr] = rows[rr] * wsp
                sdesc[k] = pltpu.async_copy(rows.at[pl.ds(k * CH, CH)],
                                            out_sh.at[idb.at[k]], sem_sc,
                                            add=True)
                if k >= 2:
                    sdesc.pop(k - 2).wait()
            for k in sorted(sdesc):
                sdesc.pop(k).wait()

        plsc.subcore_barrier()
        pltpu.sync_copy(out_sh.at[pl.ds(row0, NPT)],
                        out_hbm.at[hd, pl.ds(row0, NPT)])
        plsc.subcore_barrier()


def kernel(x, face_types, edge_index, edge_attr, params):
    p = params
    f32 = jnp.float32
    src = edge_index[0].astype(jnp.int32)
    dst = edge_index[1].astype(jnp.int32)
    pad_e = E_PAD - E
    src_p = jnp.concatenate([src, jnp.full((pad_e,), N, jnp.int32)]).reshape(ROWS, CH)
    dst_p = jnp.concatenate([dst, jnp.full((pad_e,), N, jnp.int32)]).reshape(ROWS, CH)
    ea_t = jnp.concatenate(
        [edge_attr, jnp.zeros((pad_e, EDGE_F), f32)]).T
    x_p = jnp.concatenate([x, jnp.zeros((N_PAD - N, NODE_F), f32)], axis=0)
    ft_p = jnp.concatenate(
        [face_types.astype(jnp.int32), jnp.zeros((N_PAD - N,), jnp.int32)]
    ).reshape(N_PAD, 1)

    def fold(wm, att):
        return (wm.reshape(HID, HEADS, OUT_C) * att[None]).sum(-1)

    ae_w = jnp.concatenate(
        [fold(p["gat"][l]["W_e"], p["gat"][l]["att_e"]) for l in range(LAYERS)],
        axis=1)
    a_srcs = [fold(p["gat"][l]["W"], p["gat"][l]["att_src"]) for l in range(LAYERS)]
    a_dsts = [fold(p["gat"][l]["W"], p["gat"][l]["att_dst"]) for l in range(LAYERS)]

    z4 = jnp.zeros((HEADS, N_PAD), f32)
    z16 = jnp.zeros((N_PAD, 16), f32)

    hh = _node_init(x_p, ft_p, p["face_emb"], p["node_W"], p["node_b"][None])
    ae_list = _edge_logits(ea_t, p["edge_W"].T, p["edge_b"][:, None], ae_w.T)

    prev_b = jnp.zeros((1, HID), f32)
    for l in range(LAYERS):
        g = p["gat"][l]
        xhh, asrc_t, adst_t = _layer_proj(hh, prev_b, g["W"], a_srcs[l], a_dsts[l])
        ex, den = _sc_pass_a(src_p, dst_p, ae_list[l], asrc_t, adst_t, z4)
        hh = _sc_pass_b(src_p, dst_p, ex, den, xhh, z16)
        prev_b = g["b"][None]

    hcat = jnp.concatenate([hh[k] for k in range(HEADS)], axis=1)
    mu, lv = _final(hcat, prev_b, p["mu_W"], p["mu_b"][None],
                    p["lv_W"], p["lv_b"][None])
    return (mu, lv)
